# baseline (speedup 1.0000x reference)
import functools

import jax
import jax.numpy as jnp
from jax import lax
from jax.experimental import pallas as pl
from jax.experimental.pallas import tpu as pltpu
from jax.experimental.pallas import tpu_sc as plsc

EMBED = 16
CHUNK = 128
K = 13
NC, NS = 2, 16
NW = NC * NS


@functools.lru_cache(maxsize=None)
def _build(total):
    rows = total // CHUNK
    rows_per_w = rows // NW
    n_groups = rows_per_w // K
    mesh = plsc.VectorSubcoreMesh(core_axis_name="c", subcore_axis_name="s")

    @functools.partial(
        pl.kernel,
        mesh=mesh,
        out_type=jax.ShapeDtypeStruct((total, EMBED), jnp.float32),
        scratch_types=[
            pltpu.VMEM((rows_per_w, CHUNK), jnp.int32),
            pltpu.VMEM((K, CHUNK, EMBED), jnp.float32),
            pltpu.SemaphoreType.DMA,
            pltpu.SemaphoreType.DMA,
        ],
        compiler_params=pltpu.CompilerParams(use_tc_tiling_on_sc=False),
    )
    def emb(idx_hbm, table_hbm, out_hbm, idx_v, bufs, gsem, ssem):
        wid = lax.axis_index("s") * NC + lax.axis_index("c")
        base_row = wid * rows_per_w
        pltpu.sync_copy(idx_hbm.at[pl.ds(base_row, rows_per_w)], idx_v)

        def group(g, carry):
            gathers = []
            for k in range(K):
                gathers.append(
                    pltpu.async_copy(
                        table_hbm.at[idx_v.at[g * K + k]], bufs.at[k], gsem))
            for h in gathers:
                h.wait()
            stores = []
            for k in range(K):
                stores.append(
                    pltpu.async_copy(
                        bufs.at[k],
                        out_hbm.at[pl.ds((base_row + g * K + k) * CHUNK, CHUNK)],
                        ssem))
            for h in stores:
                h.wait()
            return carry

        lax.fori_loop(0, n_groups, group, 0)

    return emb


def kernel(x, weight):
    total = x.shape[0] * x.shape[1]
    idx2d = x.reshape(total // CHUNK, CHUNK).astype(jnp.int32)
    out = _build(total)(idx2d, weight)
    return out.reshape(x.shape[0], x.shape[1], EMBED)


# speedup vs baseline: 1.5341x; 1.5341x over previous
import functools

import jax
import jax.numpy as jnp
from jax import lax
from jax.experimental import pallas as pl
from jax.experimental.pallas import tpu as pltpu
from jax.experimental.pallas import tpu_sc as plsc

EMBED = 16
CHUNK = 128
BC = 1024
RPU = BC // CHUNK
NC, NS = 2, 16
NW = NC * NS


@functools.lru_cache(maxsize=None)
def _build(batch, fields):
    cpf = batch // BC
    units_total = fields * cpf
    units_per_w = units_total // NW
    mesh = plsc.VectorSubcoreMesh(core_axis_name="c", subcore_axis_name="s")

    @functools.partial(
        pl.kernel,
        mesh=mesh,
        out_type=jax.ShapeDtypeStruct((fields, EMBED, batch), jnp.float32),
        scratch_types=[
            pltpu.VMEM((RPU, CHUNK), jnp.int32),
            pltpu.VMEM((BC, EMBED), jnp.float32),
            pltpu.VMEM((2, EMBED, BC), jnp.float32),
            pltpu.SemaphoreType.DMA,
            pltpu.SemaphoreType.DMA,
        ],
        compiler_params=pltpu.CompilerParams(
            use_tc_tiling_on_sc=False, needs_layout_passes=False),
    )
    def emb(idx_hbm, table_hbm, out_hbm, idx_v, rows_v, vals_v, gsem, ssem):
        wid = lax.axis_index("s") * NC + lax.axis_index("c")
        iota = lax.iota(jnp.int32, 16)

        def unit(u, carry):
            p = lax.rem(u, 2)
            gid = wid * units_per_w + u
            f = gid // cpf
            c = gid % cpf

            @pl.when(u >= 2)
            def _drain():
                for e in range(EMBED):
                    pltpu.make_async_copy(
                        vals_v.at[p, e],
                        out_hbm.at[0, 0, pl.ds(0, BC)],
                        ssem).wait()

            pltpu.sync_copy(idx_hbm.at[f, pl.ds(c * RPU, RPU)], idx_v)
            gathers = []
            for r in range(RPU):
                gathers.append(
                    pltpu.async_copy(
                        table_hbm.at[idx_v.at[r]],
                        rows_v.at[pl.ds(r * CHUNK, CHUNK), :],
                        gsem))
            for h in gathers:
                h.wait()

            def eloop(e, c2):
                cid = jnp.zeros((16,), jnp.int32) + e

                def vloop(vb, c3):
                    for vj in range(8):
                        v = vb * 8 + vj
                        rid = v * 16 + iota
                        vec = plsc.load_gather(rows_v, [rid, cid])
                        vals_v[p, e, pl.ds(v * 16, 16)] = vec
                    return c3

                lax.fori_loop(0, BC // (16 * 8), vloop, 0)
                return c2

            lax.fori_loop(0, EMBED, eloop, 0)

            for e in range(EMBED):
                pltpu.async_copy(
                    vals_v.at[p, e],
                    out_hbm.at[f, e, pl.ds(c * BC, BC)],
                    ssem)
            return carry

        lax.fori_loop(0, units_per_w, unit, 0)
        for q in range(2):
            for e in range(EMBED):
                pltpu.make_async_copy(
                    vals_v.at[q, e],
                    out_hbm.at[0, 0, pl.ds(0, BC)],
                    ssem).wait()

    return emb


def kernel(x, weight):
    batch, fields = x.shape
    xt3 = x.T.reshape(fields, batch // CHUNK, CHUNK).astype(jnp.int32)
    o3 = _build(batch, fields)(xt3, weight)
    return o3.transpose(2, 0, 1)


# speedup vs baseline: 1.5343x; 1.0002x over previous
import functools

import jax
import jax.numpy as jnp
from jax import lax
from jax.experimental import pallas as pl
from jax.experimental.pallas import tpu as pltpu
from jax.experimental.pallas import tpu_sc as plsc

EMBED = 16
CHUNK = 128
BC = 1024
RPU = BC // CHUNK
NC, NS = 2, 16
NW = NC * NS


@functools.lru_cache(maxsize=None)
def _build(batch, fields):
    cpf = batch // BC
    units_total = fields * cpf
    units_per_w = units_total // NW
    mesh = plsc.VectorSubcoreMesh(core_axis_name="c", subcore_axis_name="s")

    @functools.partial(
        pl.kernel,
        mesh=mesh,
        out_type=jax.ShapeDtypeStruct((fields, EMBED, batch), jnp.float32),
        scratch_types=[
            pltpu.VMEM((BC,), jnp.int32),
            pltpu.VMEM((BC, EMBED), jnp.float32),
            pltpu.VMEM((2, EMBED, BC), jnp.float32),
            pltpu.SemaphoreType.DMA,
            pltpu.SemaphoreType.DMA,
        ],
        compiler_params=pltpu.CompilerParams(
            use_tc_tiling_on_sc=False, needs_layout_passes=False),
    )
    def emb(idx_hbm, table_hbm, out_hbm, idx_v, rows_v, vals_v, gsem, ssem):
        wid = lax.axis_index("s") * NC + lax.axis_index("c")
        iota = lax.iota(jnp.int32, 16)
        iv = [vj * 16 + iota for vj in range(8)]

        def unit(u, carry):
            p = lax.rem(u, 2)
            gid = wid * units_per_w + u
            f = gid // cpf
            c = gid % cpf

            @pl.when(u >= 2)
            def _drain():
                for e in range(EMBED):
                    pltpu.make_async_copy(
                        vals_v.at[p, e],
                        out_hbm.at[0, 0, pl.ds(0, BC)],
                        ssem).wait()

            pltpu.sync_copy(idx_hbm.at[f, pl.ds(c * BC, BC)], idx_v)
            gathers = []
            for r in range(RPU):
                gathers.append(
                    pltpu.async_copy(
                        table_hbm.at[idx_v.at[pl.ds(r * CHUNK, CHUNK)]],
                        rows_v.at[pl.ds(r * CHUNK, CHUNK), :],
                        gsem))
            for h in gathers:
                h.wait()

            def vloop(vb, c3):
                base = vb * 128
                for e in range(EMBED):
                    cid = jnp.full((16,), e, jnp.int32)
                    for vj in range(8):
                        vec = plsc.load_gather(rows_v, [base + iv[vj], cid])
                        vals_v[p, e, pl.ds(base + vj * 16, 16)] = vec
                return c3

            lax.fori_loop(0, BC // 128, vloop, 0)

            for e in range(EMBED):
                pltpu.async_copy(
                    vals_v.at[p, e],
                    out_hbm.at[f, e, pl.ds(c * BC, BC)],
                    ssem)
            return carry

        lax.fori_loop(0, units_per_w, unit, 0)
        for q in range(2):
            for e in range(EMBED):
                pltpu.make_async_copy(
                    vals_v.at[q, e],
                    out_hbm.at[0, 0, pl.ds(0, BC)],
                    ssem).wait()

    return emb


def kernel(x, weight):
    batch, fields = x.shape
    xt = x.T.astype(jnp.int32)
    o3 = _build(batch, fields)(xt, weight)
    return o3.transpose(2, 0, 1)


# speedup vs baseline: 1.5860x; 1.0336x over previous
import functools

import jax
import jax.numpy as jnp
from jax import lax
from jax.experimental import pallas as pl
from jax.experimental.pallas import tpu as pltpu
from jax.experimental.pallas import tpu_sc as plsc

EMBED = 16
CHUNK = 128
BC = 1024
RPU = BC // CHUNK
NC, NS = 2, 16
NW = NC * NS


@functools.lru_cache(maxsize=None)
def _build(batch, fields):
    cpf = batch // BC
    units_per_w = fields * cpf // NW
    mesh = plsc.VectorSubcoreMesh(core_axis_name="c", subcore_axis_name="s")

    @functools.partial(
        pl.kernel,
        mesh=mesh,
        out_type=jax.ShapeDtypeStruct((fields, EMBED, batch), jnp.float32),
        scratch_types=[
            pltpu.VMEM((2, BC), jnp.float32),
            pltpu.VMEM((2, BC), jnp.int32),
            pltpu.VMEM((2, BC, EMBED), jnp.float32),
            pltpu.VMEM((2, EMBED, BC), jnp.float32),
            pltpu.SemaphoreType.DMA,
            pltpu.SemaphoreType.DMA,
        ],
        compiler_params=pltpu.CompilerParams(
            use_tc_tiling_on_sc=False, needs_layout_passes=False),
    )
    def emb(idx_hbm, table_hbm, out_hbm,
            xstage_v, idx_v, rows_v, vals_v, gsem, ssem):
        wid = lax.axis_index("s") * NC + lax.axis_index("c")
        iota = lax.iota(jnp.int32, 16)
        iv = [vj * 16 + iota for vj in range(8)]

        def stage_and_fire(uidx, pp):
            gid = wid * units_per_w + uidx
            f = gid // cpf
            c = gid % cpf
            pltpu.sync_copy(idx_hbm.at[f, pl.ds(c * BC, BC)], xstage_v.at[pp])
            for k in range(BC // 16):
                idx_v[pp, pl.ds(k * 16, 16)] = plsc.bitcast(
                    xstage_v[pp, pl.ds(k * 16, 16)], jnp.int32)
            for r in range(RPU):
                pltpu.async_copy(
                    table_hbm.at[idx_v.at[pp, pl.ds(r * CHUNK, CHUNK)]],
                    rows_v.at[pp, pl.ds(r * CHUNK, CHUNK), :],
                    gsem)

        stage_and_fire(0, 0)

        def unit(u, carry):
            pc = lax.rem(u, 2)
            gid = wid * units_per_w + u
            f = gid // cpf
            c = gid % cpf

            @pl.when(u >= 2)
            def _drain_stores():
                for e in range(EMBED):
                    pltpu.make_async_copy(
                        vals_v.at[pc, e],
                        out_hbm.at[0, 0, pl.ds(0, BC)],
                        ssem).wait()

            for r in range(RPU):
                pltpu.make_async_copy(
                    table_hbm.at[idx_v.at[pc, pl.ds(0, CHUNK)]],
                    rows_v.at[pc, pl.ds(r * CHUNK, CHUNK), :],
                    gsem).wait()

            @pl.when(u + 1 < units_per_w)
            def _prefetch():
                stage_and_fire(u + 1, 1 - pc)

            def vloop(vb, c3):
                base = vb * 128
                for e in range(EMBED):
                    cid = jnp.full((16,), e, jnp.int32)
                    for vj in range(8):
                        vec = plsc.load_gather(
                            rows_v.at[pc], [base + iv[vj], cid])
                        vals_v[pc, e, pl.ds(base + vj * 16, 16)] = vec
                return c3

            lax.fori_loop(0, BC // 128, vloop, 0)

            for e in range(EMBED):
                pltpu.async_copy(
                    vals_v.at[pc, e],
                    out_hbm.at[f, e, pl.ds(c * BC, BC)],
                    ssem)
            return carry

        lax.fori_loop(0, units_per_w, unit, 0)
        for q in range(2):
            for e in range(EMBED):
                pltpu.make_async_copy(
                    vals_v.at[q, e],
                    out_hbm.at[0, 0, pl.ds(0, BC)],
                    ssem).wait()

    return emb


def kernel(x, weight):
    batch, fields = x.shape
    xt = lax.bitcast_convert_type(x.astype(jnp.int32), jnp.float32).T
    o3 = _build(batch, fields)(xt, weight)
    return o3.transpose(2, 0, 1)
